# baseline (speedup 1.0000x reference)
import functools

import jax
import jax.numpy as jnp
import numpy as np
from jax import lax
from jax.experimental import pallas as pl
from jax.experimental.pallas import tpu as pltpu
from jax.experimental.pallas import tpu_sc as plsc

_load_gather = plsc.load_gather

_NPTS = 262144
_T = 524288
_NLV = 16
_NL = [16, 21, 28, 37, 49, 65, 86, 113, 150, 198, 261, 345, 456, 602, 794, 1048]
_PI2 = np.int32(np.int64(2654435761) - (1 << 32))
_PI3 = np.int32(805459861)

_NC, _NS = 2, 16
_NW = _NC * _NS
_PPW = _NPTS // _NW
_CHUNK = 128
_NB = 8
_SLABS = (_PPW // 16) * _NLV


def _lvl_f32(i_s, vals):
    i16 = jnp.full((16,), i_s, jnp.int32)
    out = jnp.full((16,), np.float32(vals[0]), jnp.float32)
    for k in range(1, len(vals)):
        out = jnp.where(i16 == k, jnp.full((16,), np.float32(vals[k]), jnp.float32), out)
    return out


def _worker_id():
    return lax.axis_index("s") * _NC + lax.axis_index("c")


def _sc_body(xT_hbm, tab_hbm, feat_hbm, x_v, feat_v, w_v, rows_v, sem, *idx_bufs):
    wid = _worker_id()
    wbase = wid * _PPW
    pltpu.sync_copy(xT_hbm.at[:, pl.ds(wbase, _PPW)], x_v)

    mask19 = jnp.int32(_T - 1)

    c128 = jnp.int32(_CHUNK)
    c8 = jnp.int32(8)

    def compute_fire(t, b):
        r = lax.rem(t, c128)
        i_s = lax.div(r, c8)
        g_s = lax.rem(r, c8)
        c_s = lax.div(t, c128)
        p = c_s * _CHUNK + g_s * 16
        n16 = _lvl_f32(i_s, _NL)
        off16 = jnp.full((16,), i_s, jnp.int32) * jnp.int32(_T)

        fls, frs, fis = [], [], []
        for dim in range(3):
            xv = x_v[dim, pl.ds(p, 16)]
            xm = xv * jnp.float32(0.5) + jnp.float32(0.5)
            xs = xm * n16
            fi = xs.astype(jnp.int32)
            frs.append(xs - fi.astype(jnp.float32))
            fis.append(fi)

        fx0, fy0, fz0 = fis
        fx1 = fx0 + jnp.int32(1)
        hy0 = fy0 * _PI2
        hy1 = hy0 + _PI2
        hz0 = fz0 * _PI3
        hz1 = hz0 + _PI3
        cyz = [hy0 ^ hz0, hy1 ^ hz0, hy0 ^ hz1, hy1 ^ hz1]
        one = jnp.float32(1.0)
        wx = [one - frs[0], frs[0]]
        wy = [one - frs[1], frs[1]]
        wz = [one - frs[2], frs[2]]
        wyz = [wy[0] * wz[0], wy[1] * wz[0], wy[0] * wz[1], wy[1] * wz[1]]
        vxs = [fx0, fx1]
        for v in range(8):
            bx, byz = v & 1, v >> 1
            hv = ((vxs[bx] ^ cyz[byz]) & mask19) + off16
            idx_bufs[b][pl.ds(16 * v, 16)] = hv
            w_v[jnp.int32(b), jnp.int32(v), :] = wx[bx] * wyz[byz]
        pltpu.async_copy(tab_hbm.at[idx_bufs[b]], rows_v.at[jnp.int32(b)], sem)

    def accumulate(t, b):
        r = lax.rem(t, c128)
        i_s = lax.div(r, c8)
        g_s = lax.rem(r, c8)
        pltpu.make_async_copy(tab_hbm.at[idx_bufs[b]], rows_v.at[jnp.int32(b)], sem).wait()
        acc0 = jnp.zeros((16,), jnp.float32)
        acc1 = jnp.zeros((16,), jnp.float32)
        for v in range(8):
            rowi = lax.iota(jnp.int32, 16) + jnp.int32(16 * v)
            w = w_v[jnp.int32(b), jnp.int32(v), :]
            f0 = _load_gather(rows_v.at[jnp.int32(b)], [rowi, jnp.zeros((16,), jnp.int32)])
            f1 = _load_gather(rows_v.at[jnp.int32(b)], [rowi, jnp.ones((16,), jnp.int32)])
            acc0 = acc0 + w * f0
            acc1 = acc1 + w * f1
        col = g_s * 16
        feat_v[2 * i_s, pl.ds(col, 16)] = acc0
        feat_v[2 * i_s + 1, pl.ds(col, 16)] = acc1

    for b in range(_NB):
        compute_fire(jnp.int32(b), b)

    def loop_body(w, _):
        base = w * jnp.int32(_NB)
        for b in range(_NB):
            t_acc = base + jnp.int32(b)
            accumulate(t_acc, b)
            t_new = t_acc + jnp.int32(_NB)

            @pl.when(t_new < _SLABS)
            def _():
                compute_fire(t_new, b)

            @pl.when(lax.rem(t_acc, c128) == _CHUNK - 1)
            def _():
                c_s = lax.div(t_acc, c128)
                pltpu.sync_copy(
                    feat_v, feat_hbm.at[:, pl.ds(wbase + c_s * _CHUNK, _CHUNK)])
        return jnp.int32(0)

    lax.fori_loop(jnp.int32(0), jnp.int32(_SLABS // _NB), loop_body, jnp.int32(0))


@functools.partial(jax.jit, static_argnames=())
def _sc_features(xT, tab2):
    mesh = plsc.VectorSubcoreMesh(
        core_axis_name="c", subcore_axis_name="s", num_cores=_NC, num_subcores=_NS)
    kern = pl.kernel(
        _sc_body,
        out_type=jax.ShapeDtypeStruct((2 * _NLV, _NPTS), jnp.float32),
        mesh=mesh,
        scratch_types=[
            pltpu.VMEM((3, _PPW), jnp.float32),
            pltpu.VMEM((2 * _NLV, _CHUNK), jnp.float32),
            pltpu.VMEM((_NB, 8, 16), jnp.float32),
            pltpu.VMEM((_NB, 8 * 16, 2), jnp.float32),
            pltpu.SemaphoreType.DMA,
        ] + [pltpu.VMEM((8 * 16,), jnp.int32) for _ in range(_NB)],
        compiler_params=pltpu.CompilerParams(
            needs_layout_passes=False, use_tc_tiling_on_sc=False),
    )
    return kern(xT, tab2)


_BLK = 1024


def _mlp_body(fref, dref, xref, W1r, b1r, W2r, b2r, C1r, c1r, C2r, c2r, C3r, c3r,
              oref):
    f = fref[...]
    hd1 = jnp.maximum(
        jnp.dot(W1r[...], f, preferred_element_type=jnp.float32) + b1r[...], 0.0)
    hd2 = jnp.dot(W2r[...], hd1, preferred_element_type=jnp.float32) + b2r[...]
    dm = dref[...]
    pes = [dm]
    for j in range(4):
        s = jnp.float32(2.0 ** j) * dm
        pes.append(jnp.sin(s))
        pes.append(jnp.cos(s))
    cin = jnp.concatenate([hd2] + pes, axis=0)
    h = jnp.maximum(
        jnp.dot(C1r[...], cin, preferred_element_type=jnp.float32) + c1r[...], 0.0)
    h = jnp.maximum(
        jnp.dot(C2r[...], h, preferred_element_type=jnp.float32) + c2r[...], 0.0)
    col = jax.nn.sigmoid(
        jnp.dot(C3r[...], h, preferred_element_type=jnp.float32) + c3r[...])
    xr = xref[...]
    m = (jnp.abs(xr[0:1, :]) < 1.0) & (jnp.abs(xr[1:2, :]) < 1.0) & \
        (jnp.abs(xr[2:3, :]) < 1.0)
    col = jnp.where(m, col, jnp.float32(0.0))
    sg = jnp.where(m, jnp.exp(hd2[0:1, :]), jnp.float32(0.0))
    oref[...] = jnp.concatenate([col, sg], axis=0)


def _tc_mlp(featT, dT, xT, W1, b1, W2, b2, C1, c1, C2, c2, C3, c3):
    col_spec = lambda rows: pl.BlockSpec((rows, _BLK), lambda j: (np.int32(0), j))
    full = lambda a: pl.BlockSpec(a.shape, lambda j: (np.int32(0), np.int32(0)))
    args = (W1, b1, W2, b2, C1, c1, C2, c2, C3, c3)
    return pl.pallas_call(
        _mlp_body,
        grid=(_NPTS // _BLK,),
        in_specs=[col_spec(32), col_spec(3), col_spec(3)] + [full(a) for a in args],
        out_specs=col_spec(4),
        out_shape=jax.ShapeDtypeStruct((4, _NPTS), jnp.float32),
    )(featT, dT, xT, *args)


def kernel(x, d, tables, W1, b1, W2, b2, C1, c1, C2, c2, C3, c3):
    xT = jnp.transpose(x).astype(jnp.float32)
    dT = jnp.transpose(d).astype(jnp.float32)
    tab2 = jnp.reshape(tables, (_NLV * _T, 2)).astype(jnp.float32)
    featT = _sc_features(xT, tab2)
    out4 = _tc_mlp(featT, dT, xT,
                   W1, jnp.reshape(b1, (64, 1)), W2, jnp.reshape(b2, (16, 1)),
                   C1, jnp.reshape(c1, (64, 1)), C2, jnp.reshape(c2, (64, 1)),
                   C3, jnp.reshape(c3, (3, 1)))
    color = jnp.transpose(out4[:3, :])
    sigma = out4[3, :]
    return color, sigma


# speedup vs baseline: 1.0002x; 1.0002x over previous
import functools

import jax
import jax.numpy as jnp
import numpy as np
from jax import lax
from jax.experimental import pallas as pl
from jax.experimental.pallas import tpu as pltpu
from jax.experimental.pallas import tpu_sc as plsc

_load_gather = plsc.load_gather

_NPTS = 262144
_T = 524288
_NLV = 16
_NL = [16, 21, 28, 37, 49, 65, 86, 113, 150, 198, 261, 345, 456, 602, 794, 1048]
_PI2 = np.int32(np.int64(2654435761) - (1 << 32))
_PI3 = np.int32(805459861)

_NC, _NS = 2, 16
_NW = _NC * _NS
_PPW = _NPTS // _NW
_CHUNK = 128
_NB = 8
_SLABS = (_PPW // 16) * _NLV


def _lvl_f32(i_s, vals):
    i16 = jnp.full((16,), i_s, jnp.int32)
    out = jnp.full((16,), np.float32(vals[0]), jnp.float32)
    for k in range(1, len(vals)):
        out = jnp.where(i16 == k, jnp.full((16,), np.float32(vals[k]), jnp.float32), out)
    return out


def _worker_id():
    return lax.axis_index("s") * _NC + lax.axis_index("c")


def _sc_body(xT_hbm, tab_hbm, feat_hbm, x_v, feat_v, w_v, rows_v, sem, *idx_bufs):
    wid = _worker_id()
    wbase = wid * _PPW
    pltpu.sync_copy(xT_hbm.at[:, pl.ds(wbase, _PPW)], x_v)

    mask19 = jnp.int32(_T - 1)

    c128 = jnp.int32(_CHUNK)
    c8 = jnp.int32(8)

    def compute_fire(t, b):
        r = lax.rem(t, c128)
        i_s = lax.div(r, c8)
        g_s = lax.rem(r, c8)
        c_s = lax.div(t, c128)
        p = c_s * _CHUNK + g_s * 16
        n16 = _lvl_f32(i_s, _NL)

        fls, frs, fis = [], [], []
        for dim in range(3):
            xv = x_v[dim, pl.ds(p, 16)]
            xm = xv * jnp.float32(0.5) + jnp.float32(0.5)
            xs = xm * n16
            fi = xs.astype(jnp.int32)
            frs.append(xs - fi.astype(jnp.float32))
            fis.append(fi)

        fx0, fy0, fz0 = fis
        fx1 = fx0 + jnp.int32(1)
        hy0 = fy0 * _PI2
        hy1 = hy0 + _PI2
        hz0 = fz0 * _PI3
        hz1 = hz0 + _PI3
        cyz = [hy0 ^ hz0, hy1 ^ hz0, hy0 ^ hz1, hy1 ^ hz1]
        one = jnp.float32(1.0)
        wx = [one - frs[0], frs[0]]
        wy = [one - frs[1], frs[1]]
        wz = [one - frs[2], frs[2]]
        wyz = [wy[0] * wz[0], wy[1] * wz[0], wy[0] * wz[1], wy[1] * wz[1]]
        vxs = [fx0, fx1]
        for v in range(8):
            bx, byz = v & 1, v >> 1
            hv = (vxs[bx] ^ cyz[byz]) & mask19
            idx_bufs[b][pl.ds(16 * v, 16)] = hv
            w_v[jnp.int32(b), jnp.int32(v), :] = wx[bx] * wyz[byz]
        pltpu.async_copy(tab_hbm.at[i_s].at[idx_bufs[b]], rows_v.at[jnp.int32(b)], sem)

    def accumulate(t, b):
        r = lax.rem(t, c128)
        i_s = lax.div(r, c8)
        g_s = lax.rem(r, c8)
        pltpu.make_async_copy(tab_hbm.at[i_s].at[idx_bufs[b]], rows_v.at[jnp.int32(b)], sem).wait()
        acc0 = jnp.zeros((16,), jnp.float32)
        acc1 = jnp.zeros((16,), jnp.float32)
        for v in range(8):
            rowi = lax.iota(jnp.int32, 16) + jnp.int32(16 * v)
            w = w_v[jnp.int32(b), jnp.int32(v), :]
            f0 = _load_gather(rows_v.at[jnp.int32(b)], [rowi, jnp.zeros((16,), jnp.int32)])
            f1 = _load_gather(rows_v.at[jnp.int32(b)], [rowi, jnp.ones((16,), jnp.int32)])
            acc0 = acc0 + w * f0
            acc1 = acc1 + w * f1
        col = g_s * 16
        feat_v[2 * i_s, pl.ds(col, 16)] = acc0
        feat_v[2 * i_s + 1, pl.ds(col, 16)] = acc1

    for b in range(_NB):
        compute_fire(jnp.int32(b), b)

    def loop_body(w, _):
        base = w * jnp.int32(_NB)
        for b in range(_NB):
            t_acc = base + jnp.int32(b)
            accumulate(t_acc, b)
            t_new = t_acc + jnp.int32(_NB)

            @pl.when(t_new < _SLABS)
            def _():
                compute_fire(t_new, b)

            @pl.when(lax.rem(t_acc, c128) == _CHUNK - 1)
            def _():
                c_s = lax.div(t_acc, c128)
                pltpu.sync_copy(
                    feat_v, feat_hbm.at[:, pl.ds(wbase + c_s * _CHUNK, _CHUNK)])
        return jnp.int32(0)

    lax.fori_loop(jnp.int32(0), jnp.int32(_SLABS // _NB), loop_body, jnp.int32(0))


@functools.partial(jax.jit, static_argnames=())
def _sc_features(xT, tab2):
    mesh = plsc.VectorSubcoreMesh(
        core_axis_name="c", subcore_axis_name="s", num_cores=_NC, num_subcores=_NS)
    kern = pl.kernel(
        _sc_body,
        out_type=jax.ShapeDtypeStruct((2 * _NLV, _NPTS), jnp.float32),
        mesh=mesh,
        scratch_types=[
            pltpu.VMEM((3, _PPW), jnp.float32),
            pltpu.VMEM((2 * _NLV, _CHUNK), jnp.float32),
            pltpu.VMEM((_NB, 8, 16), jnp.float32),
            pltpu.VMEM((_NB, 8 * 16, 2), jnp.float32),
            pltpu.SemaphoreType.DMA,
        ] + [pltpu.VMEM((8 * 16,), jnp.int32) for _ in range(_NB)],
        compiler_params=pltpu.CompilerParams(
            needs_layout_passes=False, use_tc_tiling_on_sc=False),
    )
    return kern(xT, tab2)


_BLK = 1024


def _mlp_body(fref, dref, xref, W1r, b1r, W2r, b2r, C1r, c1r, C2r, c2r, C3r, c3r,
              oref):
    f = fref[...]
    hd1 = jnp.maximum(
        jnp.dot(W1r[...], f, preferred_element_type=jnp.float32) + b1r[...], 0.0)
    hd2 = jnp.dot(W2r[...], hd1, preferred_element_type=jnp.float32) + b2r[...]
    dm = dref[...]
    pes = [dm]
    for j in range(4):
        s = jnp.float32(2.0 ** j) * dm
        pes.append(jnp.sin(s))
        pes.append(jnp.cos(s))
    cin = jnp.concatenate([hd2] + pes, axis=0)
    h = jnp.maximum(
        jnp.dot(C1r[...], cin, preferred_element_type=jnp.float32) + c1r[...], 0.0)
    h = jnp.maximum(
        jnp.dot(C2r[...], h, preferred_element_type=jnp.float32) + c2r[...], 0.0)
    col = jax.nn.sigmoid(
        jnp.dot(C3r[...], h, preferred_element_type=jnp.float32) + c3r[...])
    xr = xref[...]
    m = (jnp.abs(xr[0:1, :]) < 1.0) & (jnp.abs(xr[1:2, :]) < 1.0) & \
        (jnp.abs(xr[2:3, :]) < 1.0)
    col = jnp.where(m, col, jnp.float32(0.0))
    sg = jnp.where(m, jnp.exp(hd2[0:1, :]), jnp.float32(0.0))
    oref[...] = jnp.concatenate([col, sg], axis=0)


def _tc_mlp(featT, dT, xT, W1, b1, W2, b2, C1, c1, C2, c2, C3, c3):
    col_spec = lambda rows: pl.BlockSpec((rows, _BLK), lambda j: (np.int32(0), j))
    full = lambda a: pl.BlockSpec(a.shape, lambda j: (np.int32(0), np.int32(0)))
    args = (W1, b1, W2, b2, C1, c1, C2, c2, C3, c3)
    return pl.pallas_call(
        _mlp_body,
        grid=(_NPTS // _BLK,),
        in_specs=[col_spec(32), col_spec(3), col_spec(3)] + [full(a) for a in args],
        out_specs=col_spec(4),
        out_shape=jax.ShapeDtypeStruct((4, _NPTS), jnp.float32),
    )(featT, dT, xT, *args)


def kernel(x, d, tables, W1, b1, W2, b2, C1, c1, C2, c2, C3, c3):
    xT = jnp.transpose(x).astype(jnp.float32)
    dT = jnp.transpose(d).astype(jnp.float32)
    featT = _sc_features(xT, tables)
    out4 = _tc_mlp(featT, dT, xT,
                   W1, jnp.reshape(b1, (64, 1)), W2, jnp.reshape(b2, (16, 1)),
                   C1, jnp.reshape(c1, (64, 1)), C2, jnp.reshape(c2, (64, 1)),
                   C3, jnp.reshape(c3, (3, 1)))
    color = jnp.transpose(out4[:3, :])
    sigma = out4[3, :]
    return color, sigma


# speedup vs baseline: 1.2784x; 1.2782x over previous
import functools

import jax
import jax.numpy as jnp
import numpy as np
from jax import lax
from jax.experimental import pallas as pl
from jax.experimental.pallas import tpu as pltpu
from jax.experimental.pallas import tpu_sc as plsc

_load_gather = plsc.load_gather

_NPTS = 262144
_T = 524288
_NLV = 16
_NL = [16, 21, 28, 37, 49, 65, 86, 113, 150, 198, 261, 345, 456, 602, 794, 1048]
_PI2 = np.int32(np.int64(2654435761) - (1 << 32))
_PI3 = np.int32(805459861)

_NC, _NS = 2, 16
_NW = _NC * _NS
_PPW = _NPTS // _NW
_CHUNK = 128
_NB = 8
_SLABS = (_PPW // 16) * _NLV


def _lvl_f32(i_s, vals):
    i16 = jnp.full((16,), i_s, jnp.int32)
    out = jnp.full((16,), np.float32(vals[0]), jnp.float32)
    for k in range(1, len(vals)):
        out = jnp.where(i16 == k, jnp.full((16,), np.float32(vals[k]), jnp.float32), out)
    return out


def _worker_id():
    return lax.axis_index("s") * _NC + lax.axis_index("c")


_NBLK = _NLV * (_T // 128)
_BPW = _NBLK // _NW
_RCH = 32


def _relayout_body(tabF_hbm, tabP_hbm, vin, vout):
    wid = _worker_id()
    s2 = lax.iota(jnp.int32, 16) * jnp.int32(2)

    def chunk_body(ch, _):
        base = (wid * _BPW + ch * jnp.int32(_RCH)) * jnp.int32(256)
        pltpu.sync_copy(tabF_hbm.at[pl.ds(base, _RCH * 256)], vin)
        for blk in range(_RCH):
            for k in range(2):
                for g in range(8):
                    a = vin[pl.ds(blk * 256 + k * 128 + g * 16, 16)]
                    plsc.store_scatter(vout, [s2 + jnp.int32(blk * 256 + g * 32 + k)], a)
        pltpu.sync_copy(vout, tabP_hbm.at[pl.ds(base, _RCH * 256)])
        return jnp.int32(0)

    lax.fori_loop(jnp.int32(0), jnp.int32(_BPW // _RCH), chunk_body, jnp.int32(0))


def _sc_relayout(tabF):
    mesh = plsc.VectorSubcoreMesh(
        core_axis_name="c", subcore_axis_name="s", num_cores=_NC, num_subcores=_NS)
    kern = pl.kernel(
        _relayout_body,
        out_type=jax.ShapeDtypeStruct((_NLV * _T * 2,), jnp.float32),
        mesh=mesh,
        scratch_types=[
            pltpu.VMEM((_RCH * 256,), jnp.float32),
            pltpu.VMEM((_RCH * 256,), jnp.float32),
        ],
        compiler_params=pltpu.CompilerParams(
            needs_layout_passes=False, use_tc_tiling_on_sc=False),
    )
    return kern(tabF)


def _sc_body(xT_hbm, tab_hbm, feat_hbm, x_v, feat_v, w_v, rows_v, sem, *idx_bufs):
    wid = _worker_id()
    wbase = wid * _PPW
    pltpu.sync_copy(xT_hbm.at[:, pl.ds(wbase, _PPW)], x_v)

    mask19 = jnp.int32(_T - 1)

    c128 = jnp.int32(_CHUNK)
    c8 = jnp.int32(8)

    def compute_fire(t, b):
        r = lax.rem(t, c128)
        i_s = lax.div(r, c8)
        g_s = lax.rem(r, c8)
        c_s = lax.div(t, c128)
        p = c_s * _CHUNK + g_s * 16
        n16 = _lvl_f32(i_s, _NL)
        off16 = jnp.full((16,), i_s, jnp.int32) * jnp.int32(_T)

        fls, frs, fis = [], [], []
        for dim in range(3):
            xv = x_v[dim, pl.ds(p, 16)]
            xm = xv * jnp.float32(0.5) + jnp.float32(0.5)
            xs = xm * n16
            fi = xs.astype(jnp.int32)
            frs.append(xs - fi.astype(jnp.float32))
            fis.append(fi)

        fx0, fy0, fz0 = fis
        fx1 = fx0 + jnp.int32(1)
        hy0 = fy0 * _PI2
        hy1 = hy0 + _PI2
        hz0 = fz0 * _PI3
        hz1 = hz0 + _PI3
        cyz = [hy0 ^ hz0, hy1 ^ hz0, hy0 ^ hz1, hy1 ^ hz1]
        one = jnp.float32(1.0)
        wx = [one - frs[0], frs[0]]
        wy = [one - frs[1], frs[1]]
        wz = [one - frs[2], frs[2]]
        wyz = [wy[0] * wz[0], wy[1] * wz[0], wy[0] * wz[1], wy[1] * wz[1]]
        vxs = [fx0, fx1]
        for v in range(8):
            bx, byz = v & 1, v >> 1
            hv = ((vxs[bx] ^ cyz[byz]) & mask19) + off16
            idx_bufs[b][pl.ds(16 * v, 16)] = hv
            w_v[jnp.int32(b), jnp.int32(v), :] = wx[bx] * wyz[byz]
        pltpu.async_copy(tab_hbm.at[idx_bufs[b]],
                         rows_v.at[pl.ds(128 * b, 128)], sem)

    def wait_slab(b):
        pltpu.make_async_copy(tab_hbm.at[idx_bufs[b]],
                              rows_v.at[pl.ds(128 * b, 128)], sem).wait()

    def accumulate(t, b):
        r = lax.rem(t, c128)
        i_s = lax.div(r, c8)
        g_s = lax.rem(r, c8)
        acc0 = jnp.zeros((16,), jnp.float32)
        acc1 = jnp.zeros((16,), jnp.float32)
        for v in range(8):
            rowi = lax.iota(jnp.int32, 16) + jnp.int32(128 * b + 16 * v)
            w = w_v[jnp.int32(b), jnp.int32(v), :]
            f0 = _load_gather(rows_v, [rowi, jnp.zeros((16,), jnp.int32)])
            f1 = _load_gather(rows_v, [rowi, jnp.ones((16,), jnp.int32)])
            acc0 = acc0 + w * f0
            acc1 = acc1 + w * f1
        col = g_s * 16
        feat_v[2 * i_s, pl.ds(col, 16)] = acc0
        feat_v[2 * i_s + 1, pl.ds(col, 16)] = acc1

    for b in range(_NB):
        compute_fire(jnp.int32(b), b)

    def loop_body(w, _):
        base = w * jnp.int32(_NB)
        for b in range(_NB):
            t_acc = base + jnp.int32(b)
            wait_slab(b)
            accumulate(t_acc, b)
            t_new = t_acc + jnp.int32(_NB)

            @pl.when(t_new < _SLABS)
            def _():
                compute_fire(t_new, b)

            @pl.when(lax.rem(t_acc, c128) == _CHUNK - 1)
            def _():
                c_s = lax.div(t_acc, c128)
                pltpu.sync_copy(
                    feat_v, feat_hbm.at[:, pl.ds(wbase + c_s * _CHUNK, _CHUNK)])
        return jnp.int32(0)

    lax.fori_loop(jnp.int32(0), jnp.int32(_SLABS // _NB), loop_body, jnp.int32(0))


@functools.partial(jax.jit, static_argnames=())
def _sc_features(xT, tab2):
    mesh = plsc.VectorSubcoreMesh(
        core_axis_name="c", subcore_axis_name="s", num_cores=_NC, num_subcores=_NS)
    kern = pl.kernel(
        _sc_body,
        out_type=jax.ShapeDtypeStruct((2 * _NLV, _NPTS), jnp.float32),
        mesh=mesh,
        scratch_types=[
            pltpu.VMEM((3, _PPW), jnp.float32),
            pltpu.VMEM((2 * _NLV, _CHUNK), jnp.float32),
            pltpu.VMEM((_NB, 8, 16), jnp.float32),
            pltpu.VMEM((_NB * 8 * 16, 2), jnp.float32),
            pltpu.SemaphoreType.DMA,
        ] + [pltpu.VMEM((8 * 16,), jnp.int32) for _ in range(_NB)],
        compiler_params=pltpu.CompilerParams(
            needs_layout_passes=False, use_tc_tiling_on_sc=False),
    )
    return kern(xT, tab2)


_BLK = 1024


def _mlp_body(fref, dref, xref, W1r, b1r, W2r, b2r, C1r, c1r, C2r, c2r, C3r, c3r,
              oref):
    f = fref[...]
    hd1 = jnp.maximum(
        jnp.dot(W1r[...], f, preferred_element_type=jnp.float32) + b1r[...], 0.0)
    hd2 = jnp.dot(W2r[...], hd1, preferred_element_type=jnp.float32) + b2r[...]
    dm = dref[...]
    pes = [dm]
    for j in range(4):
        s = jnp.float32(2.0 ** j) * dm
        pes.append(jnp.sin(s))
        pes.append(jnp.cos(s))
    cin = jnp.concatenate([hd2] + pes, axis=0)
    h = jnp.maximum(
        jnp.dot(C1r[...], cin, preferred_element_type=jnp.float32) + c1r[...], 0.0)
    h = jnp.maximum(
        jnp.dot(C2r[...], h, preferred_element_type=jnp.float32) + c2r[...], 0.0)
    col = jax.nn.sigmoid(
        jnp.dot(C3r[...], h, preferred_element_type=jnp.float32) + c3r[...])
    xr = xref[...]
    m = (jnp.abs(xr[0:1, :]) < 1.0) & (jnp.abs(xr[1:2, :]) < 1.0) & \
        (jnp.abs(xr[2:3, :]) < 1.0)
    col = jnp.where(m, col, jnp.float32(0.0))
    sg = jnp.where(m, jnp.exp(hd2[0:1, :]), jnp.float32(0.0))
    oref[...] = jnp.concatenate([col, sg], axis=0)


def _tc_mlp(featT, dT, xT, W1, b1, W2, b2, C1, c1, C2, c2, C3, c3):
    col_spec = lambda rows: pl.BlockSpec((rows, _BLK), lambda j: (np.int32(0), j))
    full = lambda a: pl.BlockSpec(a.shape, lambda j: (np.int32(0), np.int32(0)))
    args = (W1, b1, W2, b2, C1, c1, C2, c2, C3, c3)
    return pl.pallas_call(
        _mlp_body,
        grid=(_NPTS // _BLK,),
        in_specs=[col_spec(32), col_spec(3), col_spec(3)] + [full(a) for a in args],
        out_specs=col_spec(4),
        out_shape=jax.ShapeDtypeStruct((4, _NPTS), jnp.float32),
    )(featT, dT, xT, *args)


def kernel(x, d, tables, W1, b1, W2, b2, C1, c1, C2, c2, C3, c3):
    xT = jnp.transpose(x).astype(jnp.float32)
    dT = jnp.transpose(d).astype(jnp.float32)
    tabF = jnp.reshape(
        jnp.transpose(jnp.reshape(tables, (_NLV, _T // 128, 128, 2)), (0, 1, 3, 2)),
        (-1,))
    tab2 = jnp.reshape(_sc_relayout(tabF), (_NLV * _T, 2))
    featT = _sc_features(xT, tab2)
    out4 = _tc_mlp(featT, dT, xT,
                   W1, jnp.reshape(b1, (64, 1)), W2, jnp.reshape(b2, (16, 1)),
                   C1, jnp.reshape(c1, (64, 1)), C2, jnp.reshape(c2, (64, 1)),
                   C3, jnp.reshape(c3, (3, 1)))
    color = jnp.transpose(out4[:3, :])
    sigma = out4[3, :]
    return color, sigma


# speedup vs baseline: 6.0224x; 4.7110x over previous
import functools

import jax
import jax.numpy as jnp
import numpy as np
from jax import lax
from jax.experimental import pallas as pl
from jax.experimental.pallas import tpu as pltpu
from jax.experimental.pallas import tpu_sc as plsc

_load_gather = plsc.load_gather

_NPTS = 262144
_T = 524288
_NLV = 16
_NL = [16, 21, 28, 37, 49, 65, 86, 113, 150, 198, 261, 345, 456, 602, 794, 1048]
_PI2 = np.int32(np.int64(2654435761) - (1 << 32))
_PI3 = np.int32(805459861)

_NC, _NS = 2, 16
_NW = _NC * _NS
_PPW = _NPTS // _NW
_CHUNK = 128
_NB = 8
_SLABS = (_PPW // 16) * _NLV


def _lvl_f32(i_s, vals):
    i16 = jnp.full((16,), i_s, jnp.int32)
    out = jnp.full((16,), np.float32(vals[0]), jnp.float32)
    for k in range(1, len(vals)):
        out = jnp.where(i16 == k, jnp.full((16,), np.float32(vals[k]), jnp.float32), out)
    return out


def _worker_id():
    return lax.axis_index("s") * _NC + lax.axis_index("c")


_NBLK = _NLV * (_T // 128)
_BPW = _NBLK // _NW
_RCH = 32


def _relayout_body(tabF_hbm, tabP_hbm, vin, vout):
    wid = _worker_id()
    s2 = lax.iota(jnp.int32, 16) * jnp.int32(2)

    def chunk_body(ch, _):
        base = (wid * _BPW + ch * jnp.int32(_RCH)) * jnp.int32(256)
        pltpu.sync_copy(tabF_hbm.at[pl.ds(base, _RCH * 256)], vin)
        for blk in range(_RCH):
            for k in range(2):
                for g in range(8):
                    a = vin[pl.ds(blk * 256 + k * 128 + g * 16, 16)]
                    plsc.store_scatter(vout, [s2 + jnp.int32(blk * 256 + g * 32 + k)], a)
        pltpu.sync_copy(vout, tabP_hbm.at[pl.ds(base, _RCH * 256)])
        return jnp.int32(0)

    lax.fori_loop(jnp.int32(0), jnp.int32(_BPW // _RCH), chunk_body, jnp.int32(0))


def _sc_relayout(tabF):
    mesh = plsc.VectorSubcoreMesh(
        core_axis_name="c", subcore_axis_name="s", num_cores=_NC, num_subcores=_NS)
    kern = pl.kernel(
        _relayout_body,
        out_type=jax.ShapeDtypeStruct((_NLV * _T * 2,), jnp.float32),
        mesh=mesh,
        scratch_types=[
            pltpu.VMEM((_RCH * 256,), jnp.float32),
            pltpu.VMEM((_RCH * 256,), jnp.float32),
        ],
        compiler_params=pltpu.CompilerParams(
            needs_layout_passes=False, use_tc_tiling_on_sc=False),
    )
    return kern(tabF)


def _sc_body(xT_hbm, tab_hbm, feat_hbm, x_v, feat_v, w_v, rows_v, sem, *idx_bufs):
    col_bufs = idx_bufs[_NB:]
    idx_bufs = idx_bufs[:_NB]
    wid = _worker_id()
    wbase = wid * _PPW
    pltpu.sync_copy(xT_hbm.at[:, pl.ds(wbase, _PPW)], x_v)

    mask19 = jnp.int32(_T - 1)

    c128 = jnp.int32(_CHUNK)
    c8 = jnp.int32(8)

    def compute_fire(t, b):
        r = lax.rem(t, c128)
        i_s = lax.div(r, c8)
        g_s = lax.rem(r, c8)
        c_s = lax.div(t, c128)
        p = c_s * _CHUNK + g_s * 16
        n16 = _lvl_f32(i_s, _NL)
        off16 = jnp.full((16,), i_s, jnp.int32) * jnp.int32(_T)

        fls, frs, fis = [], [], []
        for dim in range(3):
            xv = x_v[dim, pl.ds(p, 16)]
            xm = xv * jnp.float32(0.5) + jnp.float32(0.5)
            xs = xm * n16
            fi = xs.astype(jnp.int32)
            frs.append(xs - fi.astype(jnp.float32))
            fis.append(fi)

        fx0, fy0, fz0 = fis
        fx1 = fx0 + jnp.int32(1)
        hy0 = fy0 * _PI2
        hy1 = hy0 + _PI2
        hz0 = fz0 * _PI3
        hz1 = hz0 + _PI3
        cyz = [hy0 ^ hz0, hy1 ^ hz0, hy0 ^ hz1, hy1 ^ hz1]
        one = jnp.float32(1.0)
        wx = [one - frs[0], frs[0]]
        wy = [one - frs[1], frs[1]]
        wz = [one - frs[2], frs[2]]
        wyz = [wy[0] * wz[0], wy[1] * wz[0], wy[0] * wz[1], wy[1] * wz[1]]
        vxs = [fx0, fx1]
        for v in range(8):
            bx, byz = v & 1, v >> 1
            hv = ((vxs[bx] ^ cyz[byz]) & mask19) + off16
            idx_bufs[b][pl.ds(16 * v, 16)] = lax.shift_right_logical(hv, jnp.int32(2))
            col_bufs[b][pl.ds(16 * v, 16)] = (hv & jnp.int32(3)) * jnp.int32(2)
            w_v[jnp.int32(b), jnp.int32(v), :] = wx[bx] * wyz[byz]
        pltpu.async_copy(tab_hbm.at[idx_bufs[b]],
                         rows_v.at[pl.ds(128 * b, 128)], sem)

    def wait_slab(b):
        pltpu.make_async_copy(tab_hbm.at[idx_bufs[b]],
                              rows_v.at[pl.ds(128 * b, 128)], sem).wait()

    def accumulate(t, b):
        r = lax.rem(t, c128)
        i_s = lax.div(r, c8)
        g_s = lax.rem(r, c8)
        acc0 = jnp.zeros((16,), jnp.float32)
        acc1 = jnp.zeros((16,), jnp.float32)
        for v in range(8):
            rowi = lax.iota(jnp.int32, 16) + jnp.int32(128 * b + 16 * v)
            w = w_v[jnp.int32(b), jnp.int32(v), :]
            colv = col_bufs[b][pl.ds(16 * v, 16)]
            f0 = _load_gather(rows_v, [rowi, colv])
            f1 = _load_gather(rows_v, [rowi, colv + jnp.int32(1)])
            acc0 = acc0 + w * f0
            acc1 = acc1 + w * f1
        col = g_s * 16
        feat_v[2 * i_s, pl.ds(col, 16)] = acc0
        feat_v[2 * i_s + 1, pl.ds(col, 16)] = acc1

    for b in range(_NB):
        compute_fire(jnp.int32(b), b)

    def loop_body(w, _):
        base = w * jnp.int32(_NB)
        for b in range(_NB):
            t_acc = base + jnp.int32(b)
            wait_slab(b)
            accumulate(t_acc, b)
            t_new = t_acc + jnp.int32(_NB)

            @pl.when(t_new < _SLABS)
            def _():
                compute_fire(t_new, b)

            @pl.when(lax.rem(t_acc, c128) == _CHUNK - 1)
            def _():
                c_s = lax.div(t_acc, c128)
                pltpu.sync_copy(
                    feat_v, feat_hbm.at[:, pl.ds(wbase + c_s * _CHUNK, _CHUNK)])
        return jnp.int32(0)

    lax.fori_loop(jnp.int32(0), jnp.int32(_SLABS // _NB), loop_body, jnp.int32(0))


@functools.partial(jax.jit, static_argnames=())
def _sc_features(xT, tab2):
    mesh = plsc.VectorSubcoreMesh(
        core_axis_name="c", subcore_axis_name="s", num_cores=_NC, num_subcores=_NS)
    kern = pl.kernel(
        _sc_body,
        out_type=jax.ShapeDtypeStruct((2 * _NLV, _NPTS), jnp.float32),
        mesh=mesh,
        scratch_types=[
            pltpu.VMEM((3, _PPW), jnp.float32),
            pltpu.VMEM((2 * _NLV, _CHUNK), jnp.float32),
            pltpu.VMEM((_NB, 8, 16), jnp.float32),
            pltpu.VMEM((_NB * 8 * 16, 8), jnp.float32),
            pltpu.SemaphoreType.DMA,
        ] + [pltpu.VMEM((8 * 16,), jnp.int32) for _ in range(2 * _NB)],
        compiler_params=pltpu.CompilerParams(
            needs_layout_passes=False, use_tc_tiling_on_sc=False),
    )
    return kern(xT, tab2)


_BLK = 1024


def _mlp_body(fref, dref, xref, W1r, b1r, W2r, b2r, C1r, c1r, C2r, c2r, C3r, c3r,
              oref):
    f = fref[...]
    hd1 = jnp.maximum(
        jnp.dot(W1r[...], f, preferred_element_type=jnp.float32) + b1r[...], 0.0)
    hd2 = jnp.dot(W2r[...], hd1, preferred_element_type=jnp.float32) + b2r[...]
    dm = dref[...]
    pes = [dm]
    for j in range(4):
        s = jnp.float32(2.0 ** j) * dm
        pes.append(jnp.sin(s))
        pes.append(jnp.cos(s))
    cin = jnp.concatenate([hd2] + pes, axis=0)
    h = jnp.maximum(
        jnp.dot(C1r[...], cin, preferred_element_type=jnp.float32) + c1r[...], 0.0)
    h = jnp.maximum(
        jnp.dot(C2r[...], h, preferred_element_type=jnp.float32) + c2r[...], 0.0)
    col = jax.nn.sigmoid(
        jnp.dot(C3r[...], h, preferred_element_type=jnp.float32) + c3r[...])
    xr = xref[...]
    m = (jnp.abs(xr[0:1, :]) < 1.0) & (jnp.abs(xr[1:2, :]) < 1.0) & \
        (jnp.abs(xr[2:3, :]) < 1.0)
    col = jnp.where(m, col, jnp.float32(0.0))
    sg = jnp.where(m, jnp.exp(hd2[0:1, :]), jnp.float32(0.0))
    oref[...] = jnp.concatenate([col, sg], axis=0)


def _tc_mlp(featT, dT, xT, W1, b1, W2, b2, C1, c1, C2, c2, C3, c3):
    col_spec = lambda rows: pl.BlockSpec((rows, _BLK), lambda j: (np.int32(0), j))
    full = lambda a: pl.BlockSpec(a.shape, lambda j: (np.int32(0), np.int32(0)))
    args = (W1, b1, W2, b2, C1, c1, C2, c2, C3, c3)
    return pl.pallas_call(
        _mlp_body,
        grid=(_NPTS // _BLK,),
        in_specs=[col_spec(32), col_spec(3), col_spec(3)] + [full(a) for a in args],
        out_specs=col_spec(4),
        out_shape=jax.ShapeDtypeStruct((4, _NPTS), jnp.float32),
    )(featT, dT, xT, *args)


def kernel(x, d, tables, W1, b1, W2, b2, C1, c1, C2, c2, C3, c3):
    xT = jnp.transpose(x).astype(jnp.float32)
    dT = jnp.transpose(d).astype(jnp.float32)
    tabF = jnp.reshape(
        jnp.transpose(jnp.reshape(tables, (_NLV, _T // 128, 128, 2)), (0, 1, 3, 2)),
        (-1,))
    tab2 = jnp.reshape(_sc_relayout(tabF), (_NLV * _T // 4, 8))
    featT = _sc_features(xT, tab2)
    out4 = _tc_mlp(featT, dT, xT,
                   W1, jnp.reshape(b1, (64, 1)), W2, jnp.reshape(b2, (16, 1)),
                   C1, jnp.reshape(c1, (64, 1)), C2, jnp.reshape(c2, (64, 1)),
                   C3, jnp.reshape(c3, (3, 1)))
    color = jnp.transpose(out4[:3, :])
    sigma = out4[3, :]
    return color, sigma
